# baseline (speedup 1.0000x reference)
import functools

import jax
import jax.numpy as jnp
from jax import lax
from jax.experimental import pallas as pl
from jax.experimental.pallas import tpu as pltpu
from jax.experimental.pallas import tpu_sc as plsc

WIN = 64
NSLOT = 8
WC = 8
SR = 16


@functools.lru_cache(maxsize=None)
def _build_call(D, T, F, B):
    info = plsc.get_sparse_core_info()
    NC, NS, L = info.num_cores, info.num_subcores, info.num_lanes
    NW = NC * NS
    assert L == 16 and F == 64 and T % 8 == 0
    assert B % (NW * 128) == 0

    QPW = B // NW
    NBT = QPW // 128
    NCH = WIN // WC
    FT = F // 8

    mesh = plsc.VectorSubcoreMesh(core_axis_name="c", subcore_axis_name="s")

    scratch = dict(
        d_v=pltpu.VMEM((QPW,), jnp.int32),
        t_v=pltpu.VMEM((QPW,), jnp.int32),
        sh_v=pltpu.VMEM((L,), jnp.int32),
        slab=pltpu.VMEM((WC, FT, 8, 128), jnp.float32),
        d_s=pltpu.SMEM((128,), jnp.int32),
        sm_s=pltpu.SMEM((128,), jnp.int32),
        ssem=pltpu.SemaphoreType.DMA,
    )
    for k in range(NSLOT):
        scratch[f"slot{k}"] = pltpu.VMEM((SR, F), jnp.float32)
        scratch[f"gsem{k}"] = pltpu.SemaphoreType.DMA

    @functools.partial(
        pl.kernel,
        mesh=mesh,
        compiler_params=pltpu.CompilerParams(
            needs_layout_passes=False, use_tc_tiling_on_sc=True),
        out_type=jax.ShapeDtypeStruct((WIN, FT, B // 128, 8, 128), jnp.float32),
        scratch_types=scratch,
    )
    def call(data, didx, tidx, shv, out, **scr):
        slots = [scr[f"slot{k}"] for k in range(NSLOT)]
        gsems = [scr[f"gsem{k}"] for k in range(NSLOT)]
        d_v, t_v, sh_v = scr["d_v"], scr["t_v"], scr["sh_v"]
        slab, d_s, sm_s, ssem = scr["slab"], scr["d_s"], scr["sm_s"], scr["ssem"]

        wid = lax.axis_index("s") * NC + lax.axis_index("c")
        qbase = pl.multiple_of(wid * QPW, QPW)

        pltpu.sync_copy(didx.at[pl.ds(qbase, QPW)], d_v)
        pltpu.sync_copy(tidx.at[pl.ds(qbase, QPW)], t_v)
        pltpu.sync_copy(shv, sh_v)

        lane = lax.iota(jnp.int32, 16)
        shift = jnp.sum(jnp.where(lane == 0, sh_v[...], 0))
        ft_pat = [2 * kk + lane // 8 for kk in range(4)]
        f8_pat = lane % 8

        def issue_gather(qloc, bt, c, k):
            d = d_s[qloc]
            sm = sm_s[qloc]
            a0c = (sm // 8) * 8 + c * WC
            nowrap = a0c + SR <= T

            @pl.when(nowrap)
            def _():
                pltpu.async_copy(
                    data.at[d, pl.ds(a0c, SR), :], slots[k], gsems[k])

            @pl.when(jnp.logical_not(nowrap))
            def _():
                b0 = jnp.where(a0c >= T, a0c - T, a0c)
                b1 = jnp.where(a0c + 8 >= T, a0c + 8 - T, a0c + 8)
                pltpu.async_copy(
                    data.at[d, pl.ds(b0, 8), :],
                    slots[k].at[pl.ds(0, 8)], gsems[k])
                pltpu.async_copy(
                    data.at[d, pl.ds(b1, 8), :],
                    slots[k].at[pl.ds(8, 8)], gsems[k])

        def rstep(r, _):
            bt = r // NCH
            c = lax.rem(r, NCH)
            btg = qbase // 128 + bt

            @pl.when(c == 0)
            def _():
                def estep(i, _):
                    q0 = pl.multiple_of(bt * 128 + i * L, L)
                    dq = d_v[pl.ds(q0, L)]
                    tq = t_v[pl.ds(q0, L)]
                    smv = lax.rem(tq + (shift - WIN), T)
                    smv = jnp.where(smv < 0, smv + T, smv)
                    for l in range(L):
                        d_s[i * L + l] = jnp.sum(jnp.where(lane == l, dq, 0))
                        sm_s[i * L + l] = jnp.sum(jnp.where(lane == l, smv, 0))
                    return 0
                lax.fori_loop(0, 128 // L, estep, 0)

            @pl.when(r > 0)
            def _():
                pltpu.make_async_copy(
                    slab, out.at[pl.ds(0, WC), :, 0], ssem).wait()

            for k in range(NSLOT):
                issue_gather(k, bt, c, k)

            def g2step(g2, _):
                for k in range(NSLOT):
                    qloc = g2 * NSLOT + k
                    pltpu.make_async_copy(
                        data.at[0, pl.ds(0, SR), :], slots[k], gsems[k]).wait()
                    off8 = lax.rem(sm_s[qloc], 8)
                    qvec = jnp.full((L,), qloc, jnp.int32)
                    for j in range(WC):
                        jvec = jnp.full((L,), j, jnp.int32)
                        row = off8 + j
                        for kk in range(4):
                            v = slots[k][row, pl.ds(16 * kk, L)]
                            plsc.store_scatter(
                                slab, [jvec, ft_pat[kk], f8_pat, qvec], v)

                    @pl.when(g2 < 128 // NSLOT - 1)
                    def _():
                        issue_gather(qloc + NSLOT, bt, c, k)
                return 0

            lax.fori_loop(0, 128 // NSLOT, g2step, 0)

            for j in range(WC):
                pltpu.async_copy(
                    slab.at[j], out.at[c * WC + j, :, btg], ssem)
            return 0

        lax.fori_loop(0, NBT * NCH, rstep, 0)
        pltpu.make_async_copy(slab, out.at[pl.ds(0, WC), :, 0], ssem).wait()

    return call


def kernel(data, date_idx, time_idx, window_size):
    D, T, F = data.shape
    B = date_idx.shape[0]
    shv = jnp.full((16,), jnp.asarray(window_size, jnp.int32) - WIN, jnp.int32)
    phys = _build_call(D, T, F, B)(data, date_idx.astype(jnp.int32),
                                   time_idx.astype(jnp.int32), shv)
    return phys.transpose(2, 4, 0, 1, 3).reshape(B, WIN, F)


# speedup vs baseline: 1.4552x; 1.4552x over previous
import functools

import jax
import jax.numpy as jnp
from jax import lax
from jax.experimental import pallas as pl
from jax.experimental.pallas import tpu as pltpu
from jax.experimental.pallas import tpu_sc as plsc

WIN = 64
NBUF = 8
PAD = 8


@functools.lru_cache(maxsize=None)
def _build_call(D, T, F, B):
    info = plsc.get_sparse_core_info()
    NC, NS, L = info.num_cores, info.num_subcores, info.num_lanes
    NW = NC * NS
    assert L == 16 and B % (NW * L) == 0 and T % 8 == 0

    QPW = B // NW
    NG = QPW // NBUF
    SR = WIN + PAD

    mesh = plsc.VectorSubcoreMesh(core_axis_name="c", subcore_axis_name="s")

    scratch = dict(
        d_v=pltpu.VMEM((QPW,), jnp.int32),
        t_v=pltpu.VMEM((QPW,), jnp.int32),
        sh_v=pltpu.VMEM((L,), jnp.int32),
    )
    for k in range(NBUF):
        scratch[f"slot{k}"] = pltpu.VMEM((SR, F), jnp.float32)
        scratch[f"gsem{k}"] = pltpu.SemaphoreType.DMA
        scratch[f"osem{k}"] = pltpu.SemaphoreType.DMA

    @functools.partial(
        pl.kernel,
        mesh=mesh,
        compiler_params=pltpu.CompilerParams(
            needs_layout_passes=False, use_tc_tiling_on_sc=True),
        out_type=jax.ShapeDtypeStruct((B, WIN, F), jnp.float32),
        scratch_types=scratch,
    )
    def call(data, didx, tidx, shv, out, **scr):
        slots = [scr[f"slot{k}"] for k in range(NBUF)]
        gsems = [scr[f"gsem{k}"] for k in range(NBUF)]
        osems = [scr[f"osem{k}"] for k in range(NBUF)]
        d_v, t_v, sh_v = scr["d_v"], scr["t_v"], scr["sh_v"]

        wid = lax.axis_index("s") * NC + lax.axis_index("c")
        qbase = pl.multiple_of(wid * QPW, QPW)

        pltpu.sync_copy(didx.at[pl.ds(qbase, QPW)], d_v)
        pltpu.sync_copy(tidx.at[pl.ds(qbase, QPW)], t_v)
        pltpu.sync_copy(shv, sh_v)

        lane = lax.iota(jnp.int32, 16)
        shift = jnp.sum(jnp.where(lane == 0, sh_v[...], 0))

        def extract(vec_ref, q):
            base = pl.multiple_of((q // L) * L, L)
            v = vec_ref[pl.ds(base, L)]
            return jnp.sum(jnp.where(lane == q % L, v, 0))

        def gstep(g, _):
            q0 = g * NBUF
            per_slot = []
            for k in range(NBUF):
                q = q0 + k
                d = extract(d_v, q)
                t = extract(t_v, q)
                sm = lax.rem(t + (shift - WIN), T)
                sm = jnp.where(sm < 0, sm + T, sm)
                a0 = (sm // 8) * 8
                off8 = sm - a0
                per_slot.append((q, off8))

                @pl.when(g > 0)
                def _():
                    pltpu.make_async_copy(
                        slots[k].at[pl.ds(0, WIN)],
                        out.at[0], osems[k]).wait()

                nowrap = a0 + SR <= T

                @pl.when(nowrap)
                def _():
                    pltpu.async_copy(
                        data.at[d, pl.ds(a0, SR), :], slots[k], gsems[k])

                @pl.when(jnp.logical_not(nowrap))
                def _():
                    for j in range(SR // 8):
                        bj = a0 + 8 * j
                        bj = jnp.where(bj >= T, bj - T, bj)
                        pltpu.async_copy(
                            data.at[d, pl.ds(bj, 8), :],
                            slots[k].at[pl.ds(8 * j, 8)], gsems[k])

            for k in range(NBUF):
                q, off8 = per_slot[k]
                pltpu.make_async_copy(
                    data.at[0, pl.ds(0, SR), :], slots[k], gsems[k]).wait()
                pltpu.async_copy(
                    slots[k].at[pl.ds(off8, WIN)],
                    out.at[qbase + q], osems[k])
            return 0

        lax.fori_loop(0, NG, gstep, 0)
        for k in range(NBUF):
            pltpu.make_async_copy(
                slots[k].at[pl.ds(0, WIN)], out.at[0], osems[k]).wait()

    return call


NSPLIT = 2


def kernel(data, date_idx, time_idx, window_size):
    D, T, F = data.shape
    B = date_idx.shape[0]
    shv = jnp.full((16,), jnp.asarray(window_size, jnp.int32) - WIN, jnp.int32)
    di = date_idx.astype(jnp.int32)
    ti = time_idx.astype(jnp.int32)
    bs = B // NSPLIT
    call = _build_call(D, T, F, bs)
    outs = [
        call(data, di[i * bs:(i + 1) * bs], ti[i * bs:(i + 1) * bs], shv)
        for i in range(NSPLIT)
    ]
    return jnp.concatenate(outs, axis=0)


# speedup vs baseline: 2.3498x; 1.6147x over previous
import functools

import jax
import jax.numpy as jnp
from jax import lax
from jax.experimental import pallas as pl
from jax.experimental.pallas import tpu as pltpu
from jax.experimental.pallas import tpu_sc as plsc

WIN = 64
NBUF = 8
PAD = 8


@functools.lru_cache(maxsize=None)
def _build_call(D, T, F, B):
    info = plsc.get_sparse_core_info()
    NC, NS, L = info.num_cores, info.num_subcores, info.num_lanes
    NW = NC * NS
    assert L == 16 and B % (NW * L) == 0 and T % 8 == 0

    QPW = B // NW
    NG = QPW // NBUF
    SR = WIN + PAD

    mesh = plsc.VectorSubcoreMesh(core_axis_name="c", subcore_axis_name="s")

    scratch = dict(
        d_v=pltpu.VMEM((QPW,), jnp.int32),
        t_v=pltpu.VMEM((QPW,), jnp.int32),
        sh_v=pltpu.VMEM((L,), jnp.int32),
    )
    for k in range(NBUF):
        scratch[f"slot{k}"] = pltpu.VMEM((SR, F), jnp.float32)
        scratch[f"gsem{k}"] = pltpu.SemaphoreType.DMA
        scratch[f"osem{k}"] = pltpu.SemaphoreType.DMA

    @functools.partial(
        pl.kernel,
        mesh=mesh,
        compiler_params=pltpu.CompilerParams(
            needs_layout_passes=False, use_tc_tiling_on_sc=True),
        out_type=jax.ShapeDtypeStruct((B * WIN, F), jnp.float32),
        scratch_types=scratch,
    )
    def call(tbl, didx, tidx, shv, out, **scr):
        slots = [scr[f"slot{k}"] for k in range(NBUF)]
        gsems = [scr[f"gsem{k}"] for k in range(NBUF)]
        osems = [scr[f"osem{k}"] for k in range(NBUF)]
        d_v, t_v, sh_v = scr["d_v"], scr["t_v"], scr["sh_v"]

        wid = lax.axis_index("s") * NC + lax.axis_index("c")
        qbase = pl.multiple_of(wid * QPW, QPW)

        pltpu.sync_copy(didx.at[pl.ds(qbase, QPW)], d_v)
        pltpu.sync_copy(tidx.at[pl.ds(qbase, QPW)], t_v)
        pltpu.sync_copy(shv, sh_v)

        lane = lax.iota(jnp.int32, 16)
        shift = jnp.sum(jnp.where(lane == 0, sh_v[...], 0))

        def extract(vec_ref, q):
            base = pl.multiple_of((q // L) * L, L)
            v = vec_ref[pl.ds(base, L)]
            return jnp.sum(jnp.where(lane == q % L, v, 0))

        def gstep(g, _):
            q0 = g * NBUF
            per_slot = []
            for k in range(NBUF):
                q = q0 + k
                d = extract(d_v, q)
                t = extract(t_v, q)
                sm = lax.rem(t + (shift - WIN), T)
                sm = jnp.where(sm < 0, sm + T, sm)
                a0 = (sm // 8) * 8
                off8 = sm - a0
                rowbase = d * T
                per_slot.append((q, off8))

                @pl.when(g > 0)
                def _():
                    pltpu.make_async_copy(
                        slots[k].at[pl.ds(0, WIN)],
                        out.at[pl.ds(0, WIN), :], osems[k]).wait()

                nowrap = a0 + SR <= T

                @pl.when(nowrap)
                def _():
                    pltpu.async_copy(
                        tbl.at[pl.ds(rowbase + a0, SR), :], slots[k], gsems[k])

                @pl.when(jnp.logical_not(nowrap))
                def _():
                    for j in range(SR // 8):
                        bj = a0 + 8 * j
                        bj = jnp.where(bj >= T, bj - T, bj)
                        pltpu.async_copy(
                            tbl.at[pl.ds(rowbase + bj, 8), :],
                            slots[k].at[pl.ds(8 * j, 8)], gsems[k])

            for k in range(NBUF):
                q, off8 = per_slot[k]
                pltpu.make_async_copy(
                    tbl.at[pl.ds(0, SR), :], slots[k], gsems[k]).wait()
                obase = pl.multiple_of((qbase + q) * WIN, WIN)
                pltpu.async_copy(
                    slots[k].at[pl.ds(off8, WIN)],
                    out.at[pl.ds(obase, WIN), :], osems[k])
            return 0

        lax.fori_loop(0, NG, gstep, 0)
        for k in range(NBUF):
            pltpu.make_async_copy(
                slots[k].at[pl.ds(0, WIN)], out.at[pl.ds(0, WIN), :],
                osems[k]).wait()

    return call


def kernel(data, date_idx, time_idx, window_size):
    D, T, F = data.shape
    B = date_idx.shape[0]
    shv = jnp.full((16,), jnp.asarray(window_size, jnp.int32) - WIN, jnp.int32)
    out = _build_call(D, T, F, B)(data.reshape(D * T, F),
                                  date_idx.astype(jnp.int32),
                                  time_idx.astype(jnp.int32), shv)
    return out.reshape(B, WIN, F)
